# baseline (speedup 1.0000x reference)
import functools

import jax
import jax.numpy as jnp
from jax import lax
from jax.experimental import pallas as pl
from jax.experimental.pallas import tpu as pltpu
from jax.experimental.pallas import tpu_sc as plsc

VOCAB = 8192
D = 8192
B = 4096

NC = 2
NS = 16
NW = NC * NS
ROWS_PER_W = B // NW
CH = 1
NCHUNK = ROWS_PER_W // CH
NBUF = 8
LA = 5
NGROUP = NCHUNK // NBUF


def _sc_gather(table, idx):
    mesh = plsc.VectorSubcoreMesh(core_axis_name="c", subcore_axis_name="s")

    @functools.partial(
        pl.kernel,
        mesh=mesh,
        out_type=jax.ShapeDtypeStruct((B, D), jnp.float32),
        scratch_types=[
            pltpu.VMEM((NCHUNK, CH), jnp.int32),
        ]
        + [pltpu.VMEM((CH, D), jnp.float32)] * NBUF
        + [pltpu.SemaphoreType.DMA] * (2 * NBUF),
    )
    def k(table_hbm, idx_hbm, out_hbm, idx_v, *bufs_sems):
        bufs = bufs_sems[:NBUF]
        gsem = bufs_sems[NBUF : 2 * NBUF]
        wsem = bufs_sems[2 * NBUF :]
        wid = lax.axis_index("s") * NC + lax.axis_index("c")
        pltpu.sync_copy(idx_hbm.at[wid], idx_v)
        base = wid * ROWS_PER_W

        def gather(i, b):
            pltpu.async_copy(table_hbm.at[idx_v.at[i]], bufs[b], gsem[b])

        def wait_gather(i, b):
            pltpu.make_async_copy(table_hbm.at[idx_v.at[i]], bufs[b], gsem[b]).wait()

        def write(i, b):
            pltpu.async_copy(bufs[b], out_hbm.at[pl.ds(base + i * CH, CH)], wsem[b])

        def wait_write(i, b):
            pltpu.make_async_copy(
                bufs[b], out_hbm.at[pl.ds(base + i * CH, CH)], wsem[b]
            ).wait()

        for i in range(LA):
            gather(i, i)

        def body(g, carry):
            i0 = NBUF * g
            for b in range(NBUF):
                i = i0 + b
                b2 = (b + LA) % NBUF
                wait_gather(i, b)
                write(i, b)
                @pl.when(i + LA < NCHUNK)
                def _(i=i, b2=b2):
                    @pl.when(i + LA >= NBUF)
                    def _():
                        wait_write(i + LA - NBUF, b2)

                    gather(i + LA, b2)

            return carry

        lax.fori_loop(0, NGROUP, body, 0)
        for i in range(NCHUNK - NBUF, NCHUNK):
            wait_write(i, i % NBUF)

    return k(table, idx)


def kernel(x, table):
    idx = x.reshape(-1).astype(jnp.int32).reshape(NW, NCHUNK, CH)
    return _sc_gather(table, idx)
